# Initial kernel scaffold; baseline (speedup 1.0000x reference)
import jax
import jax.numpy as jnp
from jax.experimental import pallas as pl


def kernel(x, edge_index, W_in, b_in, g_in, beta_in, Wtm, btm, g_tm, beta_tm, W_out, b_out):
    raise NotImplementedError("write your pallas kernel here")


# speedup vs baseline: 3.3540x; 3.3540x over previous
import functools

import jax
import jax.numpy as jnp
from jax import lax
from jax.experimental import pallas as pl
from jax.experimental.pallas import tpu as pltpu
from jax.experimental.pallas import tpu_sc as plsc

N = 10000
D = 128
C = 16
K = 16
L = 4
REP = D // C
EPS = 1e-5

NPAD = 10240
NT = NPAD + 1024
TRASH = NPAD
NC = 2
NS = 16
NW = NC * NS

GK = 128
RB = 512
GRID = NPAD // RB
DW = 128


@functools.cache
def _mesh():
    return plsc.VectorSubcoreMesh(core_axis_name="c", subcore_axis_name="s",
                                  num_cores=NC, num_subcores=NS)


def _prep_body(ept, src_hbm, dst_hbm, dstp_hbm, degp_hbm, sbuf, dbuf, obuf,
               ones, zb, acc2):
    c = lax.axis_index("c")
    s = lax.axis_index("s")
    wid = c * NS + s
    z16f = jnp.zeros((16,), jnp.float32)
    ones16 = jnp.ones((16,), jnp.float32)
    tr16 = jnp.full((16,), TRASH, jnp.int32)

    def fill(r, carry):
        for cc in range(8):
            ones[r, pl.ds(cc * 16, 16)] = ones16
        return carry
    lax.fori_loop(0, GK, fill, 0)

    def zrow(r, carry):
        for cc in range(8):
            zb[r, pl.ds(cc * 16, 16)] = z16f
        return carry
    lax.fori_loop(0, zb.shape[0], zrow, 0)
    stripe = NT // NS
    for t in range(stripe // zb.shape[0]):
        pltpu.sync_copy(zb, acc2.at[pl.ds(s * stripe + t * zb.shape[0],
                                          zb.shape[0])])
    plsc.subcore_barrier()

    def stage(g, carry):
        base = wid * ept + g * GK
        pltpu.sync_copy(src_hbm.at[pl.ds(base, GK)], sbuf)
        pltpu.sync_copy(dst_hbm.at[pl.ds(base, GK)], dbuf)

        def inner(i, c2):
            sv = sbuf[pl.ds(i * 16, 16)]
            dv = dbuf[pl.ds(i * 16, 16)]
            obuf[pl.ds(i * 16, 16)] = jnp.where(sv != dv, dv, tr16)
            return c2
        lax.fori_loop(0, GK // 16, inner, 0)
        pltpu.sync_copy(obuf, dstp_hbm.at[pl.ds(base, GK)])
        pltpu.sync_copy(ones, acc2.at[obuf], add=True)
        return carry
    lax.fori_loop(0, ept // GK, stage, 0)
    plsc.subcore_barrier()
    wstripe = NPAD // NS
    pltpu.sync_copy(acc2.at[pl.ds(s * wstripe, wstripe)],
                    degp_hbm.at[c, pl.ds(s * wstripe, wstripe)])


def _make_prep(epad):
    ept = epad // NW
    return pl.kernel(
        functools.partial(_prep_body, ept),
        out_type=[
            jax.ShapeDtypeStruct((epad,), jnp.int32),
            jax.ShapeDtypeStruct((NC, NPAD, DW), jnp.float32),
        ],
        mesh=_mesh(),
        scratch_types=[
            pltpu.VMEM((GK,), jnp.int32),
            pltpu.VMEM((GK,), jnp.int32),
            pltpu.VMEM((GK,), jnp.int32),
            pltpu.VMEM((GK, DW), jnp.float32),
            pltpu.VMEM((64, DW), jnp.float32),
            pltpu.VMEM_SHARED((NT, DW), jnp.float32),
        ],
    )


def _agg_body(ept, hp_hbm, src_hbm, dstp_hbm, macc_hbm, isrc, idst, rows, zb,
              acc, sem):
    c = lax.axis_index("c")
    s = lax.axis_index("s")
    wid = c * NS + s
    z16f = jnp.zeros((16,), jnp.float32)

    def zrow(r, carry):
        for cc in range(8):
            zb[r, pl.ds(cc * 16, 16)] = z16f
        return carry
    lax.fori_loop(0, zb.shape[0], zrow, 0)
    stripe = NT // NS
    for t in range(stripe // zb.shape[0]):
        pltpu.sync_copy(zb, acc.at[pl.ds(s * stripe + t * zb.shape[0],
                                         zb.shape[0])])
    plsc.subcore_barrier()

    def step(g, carry):
        base = wid * ept + g * GK
        pltpu.sync_copy(src_hbm.at[pl.ds(base, GK)], isrc)
        pltpu.sync_copy(dstp_hbm.at[pl.ds(base, GK)], idst)
        pltpu.async_copy(hp_hbm.at[isrc], rows, sem).wait()
        pltpu.sync_copy(rows, acc.at[idst], add=True)
        return carry
    lax.fori_loop(0, ept // GK, step, 0)
    plsc.subcore_barrier()
    wstripe = NPAD // NS
    pltpu.sync_copy(acc.at[pl.ds(s * wstripe, wstripe)],
                    macc_hbm.at[c, pl.ds(s * wstripe, wstripe)])


def _make_agg(epad):
    ept = epad // NW
    return pl.kernel(
        functools.partial(_agg_body, ept),
        out_type=jax.ShapeDtypeStruct((NC, NPAD, D), jnp.float32),
        mesh=_mesh(),
        scratch_types=[
            pltpu.VMEM((GK,), jnp.int32),
            pltpu.VMEM((GK,), jnp.int32),
            pltpu.VMEM((GK, D), jnp.float32),
            pltpu.VMEM((64, D), jnp.float32),
            pltpu.VMEM_SHARED((NT, D), jnp.float32),
            pltpu.SemaphoreType.DMA,
        ],
    )


def _t0_body(x_ref, w_ref, b_ref, g_ref, bt_ref, d0_ref, d1_ref, h_ref,
             dinv_ref):
    pid = pl.program_id(0)
    x = x_ref[...]
    h = jnp.maximum(
        jnp.dot(x, w_ref[...], preferred_element_type=jnp.float32) + b_ref[...],
        0.0)
    mu = jnp.mean(h, axis=-1, keepdims=True)
    var = jnp.mean((h - mu) ** 2, axis=-1, keepdims=True)
    ln = (h - mu) * lax.rsqrt(var + EPS) * g_ref[...] + bt_ref[...]
    rows = pid * RB + lax.broadcasted_iota(jnp.int32, (RB, 1), 0)
    h_ref[...] = jnp.where(rows < N, ln, 0.0)
    deg = d0_ref[...][:, 0:1] + d1_ref[...][:, 0:1]
    dinv_ref[...] = 1.0 / (deg + 1.0)


_t0 = pl.pallas_call(
    _t0_body,
    grid=(GRID,),
    in_specs=[
        pl.BlockSpec((RB, D), lambda i: (i, 0)),
        pl.BlockSpec((D, D), lambda i: (0, 0)),
        pl.BlockSpec((1, D), lambda i: (0, 0)),
        pl.BlockSpec((1, D), lambda i: (0, 0)),
        pl.BlockSpec((1, D), lambda i: (0, 0)),
        pl.BlockSpec((RB, DW), lambda i: (i, 0)),
        pl.BlockSpec((RB, DW), lambda i: (i, 0)),
    ],
    out_specs=[
        pl.BlockSpec((RB, D), lambda i: (i, 0)),
        pl.BlockSpec((RB, 1), lambda i: (i, 0)),
    ],
    out_shape=[
        jax.ShapeDtypeStruct((NPAD, D), jnp.float32),
        jax.ShapeDtypeStruct((NPAD, 1), jnp.float32),
    ],
)


def _layer_body(h_ref, macc_ref, dinv_ref, tm_ref, wa_ref, wb_ref, bt_ref,
                gln_ref, bln_ref, tri_ref, rm_ref, ho_ref, tmo_ref):
    pid = pl.program_id(0)
    h = h_ref[...]
    mm = macc_ref[...]
    m = (mm[0] + mm[1] + h) * dinv_ref[...]
    z = (jnp.dot(h, wa_ref[...], preferred_element_type=jnp.float32)
         + jnp.dot(m, wb_ref[...], preferred_element_type=jnp.float32)
         + bt_ref[...])
    z = z - jnp.max(z, axis=-1, keepdims=True)
    e = jnp.exp(z)
    p = e / jnp.sum(e, axis=-1, keepdims=True)
    cs = jnp.dot(p, tri_ref[...], preferred_element_type=jnp.float32)
    tm = tm_ref[...]
    rawc = tm + (1.0 - tm) * cs
    sig = jnp.dot(rawc, rm_ref[...], preferred_element_type=jnp.float32)
    hn = h * sig + m * (1.0 - sig)
    mu = jnp.mean(hn, axis=-1, keepdims=True)
    var = jnp.mean((hn - mu) ** 2, axis=-1, keepdims=True)
    ln = (hn - mu) * lax.rsqrt(var + EPS) * gln_ref[...] + bln_ref[...]
    rows = pid * RB + lax.broadcasted_iota(jnp.int32, (RB, 1), 0)
    ho_ref[...] = jnp.where(rows < N, ln, 0.0)
    tmo_ref[...] = rawc


_tlayer = pl.pallas_call(
    _layer_body,
    grid=(GRID,),
    in_specs=[
        pl.BlockSpec((RB, D), lambda i: (i, 0)),
        pl.BlockSpec((NC, RB, D), lambda i: (0, i, 0)),
        pl.BlockSpec((RB, 1), lambda i: (i, 0)),
        pl.BlockSpec((RB, C), lambda i: (i, 0)),
        pl.BlockSpec((D, C), lambda i: (0, 0)),
        pl.BlockSpec((D, C), lambda i: (0, 0)),
        pl.BlockSpec((1, C), lambda i: (0, 0)),
        pl.BlockSpec((1, D), lambda i: (0, 0)),
        pl.BlockSpec((1, D), lambda i: (0, 0)),
        pl.BlockSpec((C, C), lambda i: (0, 0)),
        pl.BlockSpec((C, D), lambda i: (0, 0)),
    ],
    out_specs=[
        pl.BlockSpec((RB, D), lambda i: (i, 0)),
        pl.BlockSpec((RB, C), lambda i: (i, 0)),
    ],
    out_shape=[
        jax.ShapeDtypeStruct((NPAD, D), jnp.float32),
        jax.ShapeDtypeStruct((NPAD, C), jnp.float32),
    ],
)


def _tout_body(h_ref, w_ref, b_ref, o_ref):
    o_ref[...] = (jnp.dot(h_ref[...], w_ref[...],
                          preferred_element_type=jnp.float32) + b_ref[...])


_tout = pl.pallas_call(
    _tout_body,
    grid=(GRID,),
    in_specs=[
        pl.BlockSpec((RB, D), lambda i: (i, 0)),
        pl.BlockSpec((D, K), lambda i: (0, 0)),
        pl.BlockSpec((1, K), lambda i: (0, 0)),
    ],
    out_specs=pl.BlockSpec((RB, K), lambda i: (i, 0)),
    out_shape=jax.ShapeDtypeStruct((NPAD, K), jnp.float32),
)


def kernel(x, edge_index, W_in, b_in, g_in, beta_in, Wtm, btm, g_tm, beta_tm,
           W_out, b_out):
    e = edge_index.shape[1]
    epad = ((e + NW * GK - 1) // (NW * GK)) * (NW * GK)
    src = jnp.pad(edge_index[0].astype(jnp.int32), (0, epad - e))
    dst = jnp.pad(edge_index[1].astype(jnp.int32), (0, epad - e))
    xp = jnp.pad(x.astype(jnp.float32), ((0, NPAD - N), (0, 0)))

    dstp, degp = _make_prep(epad)(src, dst)

    h, dinv = _t0(xp, W_in, b_in.reshape(1, D), g_in.reshape(1, D),
                  beta_in.reshape(1, D), degp[0], degp[1])
    tm = jnp.zeros((NPAD, C), jnp.float32)
    tri = jnp.triu(jnp.ones((C, C), jnp.float32))
    rm = jnp.repeat(jnp.eye(C, dtype=jnp.float32), REP, axis=1)
    agg = _make_agg(epad)
    for j in range(L):
        macc = agg(h, src, dstp)
        h, tm = _tlayer(h, macc, dinv, tm, Wtm[j, :D, :], Wtm[j, D:, :],
                        btm[j].reshape(1, C), g_tm[j].reshape(1, D),
                        beta_tm[j].reshape(1, D), tri, rm)
    out = _tout(h, W_out, b_out.reshape(1, K))
    return out[:N]
